# Initial kernel scaffold; baseline (speedup 1.0000x reference)
import jax
import jax.numpy as jnp
from jax.experimental import pallas as pl


def kernel(idx, codebook):
    raise NotImplementedError("write your pallas kernel here")


# speedup vs baseline: 1.8308x; 1.8308x over previous
import functools

import jax
import jax.numpy as jnp
from jax import lax
from jax.experimental import pallas as pl
from jax.experimental.pallas import tpu as pltpu
from jax.experimental.pallas import tpu_sc as plsc

_D = 64
_CHUNK = 128
_GROUP = 4
_G_ROWS = _CHUNK * _GROUP


@functools.cache
def _build(num_rows):
    info = plsc.get_sparse_core_info()
    nw = info.num_cores * info.num_subcores
    rows_per_worker = num_rows // nw
    chunks_per_worker = rows_per_worker // _CHUNK
    groups = rows_per_worker // _G_ROWS

    mesh = plsc.VectorSubcoreMesh(core_axis_name="c", subcore_axis_name="s")

    @functools.partial(
        pl.kernel,
        mesh=mesh,
        compiler_params=pltpu.CompilerParams(use_tc_tiling_on_sc=False),
        out_type=jax.ShapeDtypeStruct((num_rows, _D), jnp.float32),
        scratch_types=[
            pltpu.VMEM((chunks_per_worker, _CHUNK), jnp.int32),
            pltpu.VMEM((_G_ROWS, _D), jnp.float32),
            pltpu.SemaphoreType.DMA,
        ],
    )
    def gather_kernel(idx_hbm, table_hbm, out_hbm, idx_v, rows_v, gsem):
        wid = lax.axis_index("s") * info.num_cores + lax.axis_index("c")
        chunk_base = wid * chunks_per_worker
        row_base = wid * rows_per_worker
        pltpu.sync_copy(idx_hbm.at[pl.ds(chunk_base, chunks_per_worker)], idx_v)

        def grp(g, carry):
            handles = [
                pltpu.async_copy(
                    table_hbm.at[idx_v.at[g * _GROUP + t]],
                    rows_v.at[pl.ds(t * _CHUNK, _CHUNK)],
                    gsem,
                )
                for t in range(_GROUP)
            ]
            for h in handles:
                h.wait()
            pltpu.sync_copy(
                rows_v, out_hbm.at[pl.ds(row_base + g * _G_ROWS, _G_ROWS)]
            )
            return carry

        lax.fori_loop(0, groups, grp, 0)

    return gather_kernel


def kernel(idx, codebook):
    b, s = idx.shape
    num_rows = b * s
    idx2 = idx.reshape(num_rows // _CHUNK, _CHUNK).astype(jnp.int32)
    out = _build(num_rows)(idx2, codebook)
    return out.reshape(b, s, _D)


# speedup vs baseline: 1.8719x; 1.0224x over previous
import functools

import jax
import jax.numpy as jnp
from jax import lax
from jax.experimental import pallas as pl
from jax.experimental.pallas import tpu as pltpu
from jax.experimental.pallas import tpu_sc as plsc

_GROW = 8


@functools.cache
def _build(b, s, d):
    info = plsc.get_sparse_core_info()
    nw = info.num_cores * info.num_subcores
    rows_per_worker = b // nw
    groups = rows_per_worker // _GROW

    mesh = plsc.VectorSubcoreMesh(core_axis_name="c", subcore_axis_name="s")

    @functools.partial(
        pl.kernel,
        mesh=mesh,
        compiler_params=pltpu.CompilerParams(use_tc_tiling_on_sc=False),
        out_type=jax.ShapeDtypeStruct((b, s, d), jnp.float32),
        scratch_types=[
            pltpu.VMEM((rows_per_worker, s), jnp.int32),
            pltpu.VMEM((2, _GROW, s, d), jnp.float32),
            pltpu.SemaphoreType.DMA,
            pltpu.SemaphoreType.DMA,
            pltpu.SemaphoreType.DMA,
            pltpu.SemaphoreType.DMA,
        ],
    )
    def gather_kernel(idx_hbm, table_hbm, out_hbm, idx_v, rows_v,
                      gsem0, gsem1, wsem0, wsem1):
        wid = lax.axis_index("s") * info.num_cores + lax.axis_index("c")
        row0 = wid * rows_per_worker
        gsems = (gsem0, gsem1)
        wsems = (wsem0, wsem1)

        pltpu.sync_copy(idx_hbm.at[pl.ds(row0, rows_per_worker)], idx_v)

        def fire_gather(bf, g):
            for r in range(_GROW):
                pltpu.async_copy(
                    table_hbm.at[idx_v.at[g * _GROW + r]],
                    rows_v.at[bf].at[r],
                    gsems[bf],
                )

        def wait_gather(bf, g):
            for r in range(_GROW):
                pltpu.make_async_copy(
                    table_hbm.at[idx_v.at[g * _GROW + r]],
                    rows_v.at[bf].at[r],
                    gsems[bf],
                ).wait()

        def fire_wb(bf, g):
            return pltpu.async_copy(
                rows_v.at[bf],
                out_hbm.at[pl.ds(row0 + g * _GROW, _GROW)],
                wsems[bf],
            )

        fire_gather(0, 0)
        fire_gather(1, 1)

        def body(gp, carry):
            for bf in range(2):
                g = 2 * gp + bf
                wait_gather(bf, g)
                fire_wb(bf, g)
                @pl.when(gp < groups // 2 - 1)
                def _():
                    pltpu.make_async_copy(
                        rows_v.at[bf],
                        out_hbm.at[pl.ds(row0 + g * _GROW, _GROW)],
                        wsems[bf],
                    ).wait()
                    fire_gather(bf, g + 2)
            return carry

        lax.fori_loop(0, groups // 2, body, 0)

        for bf in range(2):
            g = groups - 2 + bf
            pltpu.make_async_copy(
                rows_v.at[bf],
                out_hbm.at[pl.ds(row0 + g * _GROW, _GROW)],
                wsems[bf],
            ).wait()

    return gather_kernel


def kernel(idx, codebook):
    b, s = idx.shape
    d = codebook.shape[1]
    return _build(b, s, d)(idx.astype(jnp.int32), codebook)
